# baseline (speedup 1.0000x reference)
import jax
import jax.numpy as jnp
from jax.experimental import pallas as pl
from jax.experimental.pallas import tpu as pltpu

N = 8192
TI = 2048
NI = N // TI
NC = 2
CW = N // NC


def _assd_kernel(pred_ref, realT_ref, out_ref, accrow_s, colmin_s):
    i = pl.program_id(0)

    p = pred_ref[...]
    p2 = jnp.sum(p * p, axis=1, keepdims=True)
    pm2 = -2.0 * p

    rowmin = None
    for c in range(NC):
        rT = realT_ref[:, c * CW:(c + 1) * CW]
        r2 = jnp.sum(rT * rT, axis=0, keepdims=True)
        cross2 = jax.lax.dot_general(
            pm2, rT, (((1,), (0,)), ((), ())),
            preferred_element_type=jnp.float32)
        e = cross2 + r2
        f = cross2 + p2
        cmin = jnp.min(e, axis=1, keepdims=True)
        rowmin = cmin if rowmin is None else jnp.minimum(rowmin, cmin)
        tile_colmin = jnp.min(f, axis=0, keepdims=True) + r2

        @pl.when(i == 0)
        def _():
            colmin_s[:, c * CW:(c + 1) * CW] = tile_colmin

        @pl.when(i > 0)
        def _():
            colmin_s[:, c * CW:(c + 1) * CW] = jnp.minimum(
                colmin_s[:, c * CW:(c + 1) * CW], tile_colmin)

    row_nn = jnp.sqrt(jnp.maximum(rowmin + p2, 1e-12))

    @pl.when(i == 0)
    def _():
        accrow_s[...] = row_nn

    @pl.when(i > 0)
    def _():
        accrow_s[...] = accrow_s[...] + row_nn

    @pl.when(i == NI - 1)
    def _():
        col_nn = jnp.sqrt(jnp.maximum(colmin_s[...], 1e-12))
        total_row = jnp.sum(accrow_s[...], keepdims=True)
        total_col = jnp.sum(col_nn, keepdims=True)
        out_ref[...] = (total_row + total_col) / (2.0 * N)


def kernel(real_pts, pred_pts):
    realT = real_pts.T
    out = pl.pallas_call(
        _assd_kernel,
        grid=(NI,),
        in_specs=[
            pl.BlockSpec((TI, 3), lambda i: (i, 0)),
            pl.BlockSpec((3, N), lambda i: (0, 0)),
        ],
        out_specs=pl.BlockSpec((1, 1), lambda i: (0, 0)),
        out_shape=jax.ShapeDtypeStruct((1, 1), jnp.float32),
        scratch_shapes=[
            pltpu.VMEM((TI, 1), jnp.float32),
            pltpu.VMEM((1, N), jnp.float32),
        ],
    )(pred_pts, realT)
    return out[0, 0]


# speedup vs baseline: 1.0017x; 1.0017x over previous
import jax
import jax.numpy as jnp
from jax.experimental import pallas as pl
from jax.experimental.pallas import tpu as pltpu

N = 8192
TI = 1024
NI = N // TI
NC = 1
CW = N // NC


def _assd_kernel(pred_ref, realT_ref, out_ref, accrow_s, colmin_s):
    i = pl.program_id(0)

    p = pred_ref[...]
    p2 = jnp.sum(p * p, axis=1, keepdims=True)
    pm2 = -2.0 * p

    rowmin = None
    for c in range(NC):
        rT = realT_ref[:, c * CW:(c + 1) * CW]
        r2 = jnp.sum(rT * rT, axis=0, keepdims=True)
        cross2 = jax.lax.dot_general(
            pm2, rT, (((1,), (0,)), ((), ())),
            preferred_element_type=jnp.float32)
        e = cross2 + r2
        f = cross2 + p2
        cmin = jnp.min(e, axis=1, keepdims=True)
        rowmin = cmin if rowmin is None else jnp.minimum(rowmin, cmin)
        tile_colmin = jnp.min(f, axis=0, keepdims=True) + r2

        @pl.when(i == 0)
        def _():
            colmin_s[:, c * CW:(c + 1) * CW] = tile_colmin

        @pl.when(i > 0)
        def _():
            colmin_s[:, c * CW:(c + 1) * CW] = jnp.minimum(
                colmin_s[:, c * CW:(c + 1) * CW], tile_colmin)

    row_nn = jnp.sqrt(jnp.maximum(rowmin + p2, 1e-12))

    @pl.when(i == 0)
    def _():
        accrow_s[...] = row_nn

    @pl.when(i > 0)
    def _():
        accrow_s[...] = accrow_s[...] + row_nn

    @pl.when(i == NI - 1)
    def _():
        col_nn = jnp.sqrt(jnp.maximum(colmin_s[...], 1e-12))
        total_row = jnp.sum(accrow_s[...], keepdims=True)
        total_col = jnp.sum(col_nn, keepdims=True)
        out_ref[...] = (total_row + total_col) / (2.0 * N)


def kernel(real_pts, pred_pts):
    realT = real_pts.T
    out = pl.pallas_call(
        _assd_kernel,
        grid=(NI,),
        in_specs=[
            pl.BlockSpec((TI, 3), lambda i: (i, 0)),
            pl.BlockSpec((3, N), lambda i: (0, 0)),
        ],
        out_specs=pl.BlockSpec((1, 1), lambda i: (0, 0)),
        out_shape=jax.ShapeDtypeStruct((1, 1), jnp.float32),
        scratch_shapes=[
            pltpu.VMEM((TI, 1), jnp.float32),
            pltpu.VMEM((1, N), jnp.float32),
        ],
    )(pred_pts, realT)
    return out[0, 0]
